# Initial kernel scaffold; baseline (speedup 1.0000x reference)
import jax
import jax.numpy as jnp
from jax.experimental import pallas as pl


def kernel(x, p_t, edge_index, edge_weights, W_value, b_value):
    raise NotImplementedError("write your pallas kernel here")


# speedup vs baseline: 124.9532x; 124.9532x over previous
import functools

import jax
import jax.numpy as jnp
from jax import lax
from jax.experimental import pallas as pl
from jax.experimental.pallas import tpu as pltpu
from jax.experimental.pallas import tpu_sc as plsc

_N = 10000
_E = 320000
_D = 128
_L = 16


def _sc_indegree_mask(dst):
    mesh = plsc.VectorSubcoreMesh(core_axis_name="c", subcore_axis_name="s")
    nc, ns = mesh.num_cores, mesh.num_subcores
    nw = nc * ns
    assert _E % nw == 0
    epw = _E // nw
    n_pad = ((_N + ns * _L - 1) // (ns * _L)) * (ns * _L)
    seg = n_pad // ns

    @functools.partial(
        pl.kernel,
        mesh=mesh,
        out_type=jax.ShapeDtypeStruct((nc, n_pad), jnp.float32),
        compiler_params=pltpu.CompilerParams(needs_layout_passes=False),
        scratch_types=[
            pltpu.VMEM((epw,), jnp.int32),
            pltpu.VMEM((n_pad,), jnp.float32),
            pltpu.VMEM((ns, seg), jnp.float32),
            pltpu.VMEM_SHARED((ns, ns, seg), jnp.float32),
        ],
    )
    def k(dst_hbm, out_hbm, idx_v, mask_v, merge_v, stage_s):
        c = lax.axis_index("c")
        s = lax.axis_index("s")
        wid = c * ns + s
        zero16 = jnp.zeros((_L,), jnp.float32)
        one16 = jnp.ones((_L,), jnp.float32)

        def init_body(i, carry):
            mask_v[pl.ds(i * _L, _L)] = zero16
            return carry

        lax.fori_loop(0, n_pad // _L, init_body, 0)

        pltpu.sync_copy(dst_hbm.at[pl.ds(wid * epw, epw)], idx_v)

        def scat_body(i, carry):
            idx = idx_v[pl.ds(i * _L, _L)]
            plsc.store_scatter(mask_v, [idx], one16)
            return carry

        lax.fori_loop(0, epw // _L, scat_body, 0)

        for chunk in range(ns):
            pltpu.sync_copy(mask_v.at[pl.ds(chunk * seg, seg)],
                            stage_s.at[chunk, s])
        plsc.subcore_barrier()
        pltpu.sync_copy(stage_s.at[s], merge_v)

        def merge_body(j, carry):
            acc = zero16
            for t in range(ns):
                acc = acc + merge_v[t, pl.ds(j * _L, _L)]
            mask_v[pl.ds(j * _L, _L)] = acc
            return carry

        lax.fori_loop(0, seg // _L, merge_body, 0)

        pltpu.sync_copy(mask_v.at[pl.ds(0, seg)],
                        out_hbm.at[c, pl.ds(s * seg, seg)])

    return k(dst)


def _tc_body(x_ref, wt_ref, b_ref, cnt_ref, o_ref):
    y = jnp.dot(x_ref[...], wt_ref[...], preferred_element_type=jnp.float32)
    y = jnp.maximum(y + b_ref[...], 0.0)
    cnt = cnt_ref[...]
    alive = jnp.sum(cnt, axis=1, keepdims=True) > 0.0
    o_ref[...] = jnp.where(alive, y, 0.0)


def kernel(x, p_t, edge_index, edge_weights, W_value, b_value):
    del p_t, edge_weights
    dst = edge_index[1]
    counts = _sc_indegree_mask(dst)
    cnt_t = counts.T[:_N]
    wt = W_value.T
    b2 = b_value.reshape(1, _D)
    out = pl.pallas_call(
        _tc_body,
        out_shape=jax.ShapeDtypeStruct((_N, _D), jnp.float32),
    )(x, wt, b2, cnt_t)
    return out


# speedup vs baseline: 171.2187x; 1.3703x over previous
import functools

import jax
import jax.numpy as jnp
from jax import lax
from jax.experimental import pallas as pl
from jax.experimental.pallas import tpu as pltpu
from jax.experimental.pallas import tpu_sc as plsc

_N = 10000
_E = 320000
_D = 128
_L = 16


def _sc_indegree_mask(edge_index):
    mesh = plsc.VectorSubcoreMesh(core_axis_name="c", subcore_axis_name="s")
    nc, ns = mesh.num_cores, mesh.num_subcores
    nw = nc * ns
    assert _E % (nw * _L * 5) == 0
    epw = _E // nw
    n_pad = ((_N + ns * _L - 1) // (ns * _L)) * (ns * _L)
    seg = n_pad // ns

    @functools.partial(
        pl.kernel,
        mesh=mesh,
        out_type=jax.ShapeDtypeStruct((nc, n_pad), jnp.float32),
        compiler_params=pltpu.CompilerParams(
            needs_layout_passes=False, use_tc_tiling_on_sc=False),
        scratch_types=[
            pltpu.VMEM((epw,), jnp.int32),
            pltpu.VMEM((n_pad,), jnp.float32),
            pltpu.VMEM((ns, seg), jnp.float32),
            pltpu.VMEM_SHARED((ns, ns, seg), jnp.float32),
            pltpu.SemaphoreType.DMA,
            pltpu.SemaphoreType.DMA,
        ],
    )
    def k(ei_hbm, out_hbm, idx_v, mask_v, merge_v, stage_s, idx_sem, st_sem):
        c = lax.axis_index("c")
        s = lax.axis_index("s")
        wid = c * ns + s
        zero16 = jnp.zeros((_L,), jnp.float32)
        one16 = jnp.ones((_L,), jnp.float32)

        idx_cp = pltpu.async_copy(
            ei_hbm.at[1, pl.ds(wid * epw, epw)], idx_v, idx_sem)

        def init_body(i, carry):
            for u in range(8):
                mask_v[pl.ds((i * 8 + u) * _L, _L)] = zero16
            return carry

        lax.fori_loop(0, n_pad // _L // 8, init_body, 0)
        idx_cp.wait()

        def scat_body(i, carry):
            for u in range(5):
                idx = idx_v[pl.ds((i * 5 + u) * _L, _L)]
                plsc.store_scatter(mask_v, [idx], one16)
            return carry

        lax.fori_loop(0, epw // _L // 5, scat_body, 0)

        copies = [
            pltpu.async_copy(mask_v.at[pl.ds(chunk * seg, seg)],
                             stage_s.at[chunk, s], st_sem)
            for chunk in range(ns)
        ]
        for cp in copies:
            cp.wait()
        plsc.subcore_barrier()
        pltpu.sync_copy(stage_s.at[s], merge_v)

        def merge_body(j, carry):
            acc = zero16
            for t in range(ns):
                acc = acc + merge_v[t, pl.ds(j * _L, _L)]
            mask_v[pl.ds(j * _L, _L)] = acc
            return carry

        lax.fori_loop(0, seg // _L, merge_body, 0)

        pltpu.sync_copy(mask_v.at[pl.ds(0, seg)],
                        out_hbm.at[c, pl.ds(s * seg, seg)])

    return k(edge_index)


def _tc_body(x_ref, wt_ref, b_ref, cnt_ref, o_ref):
    y = jnp.dot(x_ref[...], wt_ref[...], preferred_element_type=jnp.float32)
    y = jnp.maximum(y + b_ref[...], 0.0)
    cnt = cnt_ref[...]
    alive = jnp.sum(cnt, axis=1, keepdims=True) > 0.0
    o_ref[...] = jnp.where(alive, y, 0.0)


def kernel(x, p_t, edge_index, edge_weights, W_value, b_value):
    del p_t, edge_weights
    counts = _sc_indegree_mask(edge_index)
    cnt_t = counts.T[:_N]
    wt = W_value.T
    b2 = b_value.reshape(1, _D)
    out = pl.pallas_call(
        _tc_body,
        out_shape=jax.ShapeDtypeStruct((_N, _D), jnp.float32),
    )(x, wt, b2, cnt_t)
    return out


# speedup vs baseline: 188.7230x; 1.1022x over previous
import functools

import jax
import jax.numpy as jnp
from jax import lax
from jax.experimental import pallas as pl
from jax.experimental.pallas import tpu as pltpu
from jax.experimental.pallas import tpu_sc as plsc

_N = 10000
_E = 320000
_D = 128
_L = 16


def _sc_indegree_mask(edge_index):
    mesh = plsc.VectorSubcoreMesh(core_axis_name="c", subcore_axis_name="s")
    nc, ns = mesh.num_cores, mesh.num_subcores
    nw = nc * ns
    assert _E % (nw * _L * 5) == 0
    epw = _E // nw
    n_pad = ((_N + ns * _L - 1) // (ns * _L)) * (ns * _L)
    seg = n_pad // ns

    @functools.partial(
        pl.kernel,
        mesh=mesh,
        out_type=jax.ShapeDtypeStruct((nc, n_pad), jnp.float32),
        compiler_params=pltpu.CompilerParams(
            needs_layout_passes=False, use_tc_tiling_on_sc=False),
        scratch_types=[
            pltpu.VMEM((epw,), jnp.int32),
            pltpu.VMEM((n_pad,), jnp.float32),
            pltpu.VMEM((ns, seg), jnp.float32),
            pltpu.VMEM_SHARED((ns, ns, seg), jnp.float32),
            pltpu.SemaphoreType.DMA,
            pltpu.SemaphoreType.DMA,
        ],
    )
    def k(ei_hbm, out_hbm, idx_v, mask_v, merge_v, stage_s, idx_sem, st_sem):
        c = lax.axis_index("c")
        s = lax.axis_index("s")
        wid = c * ns + s
        zero16 = jnp.zeros((_L,), jnp.float32)
        one16 = jnp.ones((_L,), jnp.float32)

        idx_cp = pltpu.async_copy(
            ei_hbm.at[1, pl.ds(wid * epw, epw)], idx_v, idx_sem)

        def init_body(i, carry):
            for u in range(8):
                mask_v[pl.ds((i * 8 + u) * _L, _L)] = zero16
            return carry

        lax.fori_loop(0, n_pad // _L // 8, init_body, 0)
        idx_cp.wait()

        def scat_body(i, carry):
            for u in range(5):
                idx = idx_v[pl.ds((i * 5 + u) * _L, _L)]
                plsc.store_scatter(mask_v, [idx], one16)
            return carry

        lax.fori_loop(0, epw // _L // 5, scat_body, 0)

        copies = [
            pltpu.async_copy(mask_v.at[pl.ds(chunk * seg, seg)],
                             stage_s.at[chunk, s], st_sem)
            for chunk in range(ns)
        ]
        for cp in copies:
            cp.wait()
        plsc.subcore_barrier()
        pltpu.sync_copy(stage_s.at[s], merge_v)

        def merge_body(j, carry):
            acc = zero16
            for t in range(ns):
                acc = acc + merge_v[t, pl.ds(j * _L, _L)]
            mask_v[pl.ds(j * _L, _L)] = acc
            return carry

        lax.fori_loop(0, seg // _L, merge_body, 0)

        pltpu.sync_copy(mask_v.at[pl.ds(0, seg)],
                        out_hbm.at[c, pl.ds(s * seg, seg)])

    return k(edge_index)


def _tc_body(x_ref, wt_ref, b_ref, cnt_ref, o_ref):
    y = jnp.dot(x_ref[...], wt_ref[...], preferred_element_type=jnp.float32)
    y = jnp.maximum(y + b_ref[...], 0.0)
    cnt = cnt_ref[...]
    alive_row = (cnt[0:1, :_N] + cnt[1:2, :_N]) > 0.0
    alive = jnp.transpose(alive_row)
    o_ref[...] = jnp.where(alive, y, 0.0)


def kernel(x, p_t, edge_index, edge_weights, W_value, b_value):
    del p_t, edge_weights
    counts = _sc_indegree_mask(edge_index)
    wt = W_value.T
    b2 = b_value.reshape(1, _D)
    out = pl.pallas_call(
        _tc_body,
        out_shape=jax.ShapeDtypeStruct((_N, _D), jnp.float32),
    )(x, wt, b2, counts)
    return out


# speedup vs baseline: 189.6236x; 1.0048x over previous
import functools

import jax
import jax.numpy as jnp
from jax import lax
from jax.experimental import pallas as pl
from jax.experimental.pallas import tpu as pltpu
from jax.experimental.pallas import tpu_sc as plsc

_N = 10000
_E = 320000
_D = 128
_L = 16


def _sc_indegree_mask(edge_index):
    mesh = plsc.VectorSubcoreMesh(core_axis_name="c", subcore_axis_name="s")
    nc, ns = mesh.num_cores, mesh.num_subcores
    nw = nc * ns
    assert _E % (nw * _L * 5) == 0
    epw = _E // nw
    n_pad = ((_N + ns * _L - 1) // (ns * _L)) * (ns * _L)
    seg = n_pad // ns

    @functools.partial(
        pl.kernel,
        mesh=mesh,
        out_type=jax.ShapeDtypeStruct((nc, n_pad), jnp.float32),
        compiler_params=pltpu.CompilerParams(
            needs_layout_passes=False, use_tc_tiling_on_sc=False),
        scratch_types=[
            pltpu.VMEM((epw,), jnp.int32),
            pltpu.VMEM((n_pad,), jnp.float32),
            pltpu.VMEM((ns, seg), jnp.float32),
            pltpu.VMEM_SHARED((ns, ns, seg), jnp.float32),
            pltpu.SemaphoreType.DMA,
            pltpu.SemaphoreType.DMA,
        ],
    )
    def k(ei_hbm, out_hbm, idx_v, mask_v, merge_v, stage_s, idx_sem, st_sem):
        c = lax.axis_index("c")
        s = lax.axis_index("s")
        wid = c * ns + s
        zero16 = jnp.zeros((_L,), jnp.float32)
        one16 = jnp.ones((_L,), jnp.float32)

        idx_cp = pltpu.async_copy(
            ei_hbm.at[1, pl.ds(wid * epw, epw)], idx_v, idx_sem)

        def init_body(i, carry):
            for u in range(8):
                mask_v[pl.ds((i * 8 + u) * _L, _L)] = zero16
            return carry

        lax.fori_loop(0, n_pad // _L // 8, init_body, 0)
        idx_cp.wait()

        def scat_body(i, carry):
            for u in range(5):
                idx = idx_v[pl.ds((i * 5 + u) * _L, _L)]
                plsc.store_scatter(mask_v, [idx], one16)
            return carry

        lax.fori_loop(0, epw // _L // 5, scat_body, 0)

        copies = [
            pltpu.async_copy(mask_v.at[pl.ds(chunk * seg, seg)],
                             stage_s.at[chunk, s], st_sem)
            for chunk in range(ns)
        ]
        for cp in copies:
            cp.wait()
        plsc.subcore_barrier()
        pltpu.sync_copy(stage_s.at[s], merge_v)

        def merge_body(j, carry):
            acc = zero16
            for t in range(ns):
                acc = acc + merge_v[t, pl.ds(j * _L, _L)]
            mask_v[pl.ds(j * _L, _L)] = acc
            return carry

        lax.fori_loop(0, seg // _L, merge_body, 0)

        pltpu.sync_copy(mask_v.at[pl.ds(0, seg)],
                        out_hbm.at[c, pl.ds(s * seg, seg)])

    return k(edge_index)


def _tc_body(x_ref, w_ref, b_ref, cnt_ref, o_ref):
    y = jax.lax.dot_general(
        x_ref[...], w_ref[...], (((1,), (1,)), ((), ())),
        preferred_element_type=jnp.float32)
    y = jnp.maximum(y + b_ref[...][None, :], 0.0)
    cnt = cnt_ref[...]
    alive_row = (cnt[0:1, :_N] + cnt[1:2, :_N]) > 0.0
    alive = jnp.transpose(alive_row)
    o_ref[...] = jnp.where(alive, y, 0.0)


def kernel(x, p_t, edge_index, edge_weights, W_value, b_value):
    del p_t, edge_weights
    counts = _sc_indegree_mask(edge_index)
    out = pl.pallas_call(
        _tc_body,
        out_shape=jax.ShapeDtypeStruct((_N, _D), jnp.float32),
    )(x, W_value, b_value, counts)
    return out
